# Initial kernel scaffold; baseline (speedup 1.0000x reference)
import jax
import jax.numpy as jnp
from jax.experimental import pallas as pl


def kernel(X, part_masks, all_h_t, all_c_t, W_in, b_in, W_soc, b_soc, W_ih, W_hh, b_ih, b_hh, W_out, b_out, T_obs, T_pred):
    raise NotImplementedError("write your pallas kernel here")


# speedup vs baseline: 38.0683x; 38.0683x over previous
import jax
import jax.numpy as jnp
from jax.experimental import pallas as pl
from jax.experimental.pallas import tpu as pltpu

_N_SIZE = 4
_CELL = 1.0


def _social_lstm_body(tob_ref, tpr_ref, xt_ref, xc_ref, pm_ref, h0_ref, c0_ref,
                      win_ref, bin_ref, wsoc_ref, bsoc_ref, wih_ref, whh_ref,
                      bih_ref, bhh_ref, wout_ref, bout_ref,
                      out_ref, h_scr, c_scr):
    T = out_ref.shape[0]
    N = out_ref.shape[2]
    med = win_ref.shape[0]
    hid = whh_ref.shape[1]
    ncell = _N_SIZE * _N_SIZE
    half = _N_SIZE / 2.0

    out_ref[:] = jnp.zeros(out_ref.shape, out_ref.dtype)
    h_scr[:] = h0_ref[:]
    c_scr[:] = c0_ref[:]

    wsoc = wsoc_ref[:]
    b_eff = bsoc_ref[:] + jnp.sum(wsoc, axis=1, keepdims=True)
    b_lstm = bih_ref[:] + bhh_ref[:]
    wih_r = wih_ref[:, 0:med]
    wih_e = wih_ref[:, med:]
    whh = whh_ref[:]
    wout = wout_ref[:]
    win = win_ref[:]
    bin_c = bin_ref[:]
    bout_c = bout_ref[:]

    iota_l = jax.lax.broadcasted_iota(jnp.int32, (N, N), 1)
    iota_s = jax.lax.broadcasted_iota(jnp.int32, (N, N), 0)
    noteye = iota_l != iota_s

    def step(f, carry):
        xt = xt_ref[pl.ds(f, 1)][0]
        xc = xc_ref[pl.ds(f, 1)][0]
        pmf = pm_ref[pl.ds(f, 1)][0]
        hT = h_scr[:]
        cT = c_scr[:]

        cx_r = xt[2:3, :]
        cy_r = xt[3:4, :]
        cx_c = xc[:, 0:1]
        cy_c = xc[:, 1:2]

        gx = (cx_c - cx_r) / _CELL
        gy = (cy_c - cy_r) / _CELL
        inb = (jnp.abs(gx) <= half) & (jnp.abs(gy) <= half)
        mask = inb & noteye
        ix = jnp.clip(jnp.floor(gx + half).astype(jnp.int32), 0, _N_SIZE - 1)
        iy = jnp.clip(jnp.floor(gy + half).astype(jnp.int32), 0, _N_SIZE - 1)
        cell = ix * _N_SIZE + iy

        rT = jax.nn.relu(
            jnp.dot(win, xt[2:4, :], preferred_element_type=jnp.float32)
            + bin_c)

        slabs = []
        for c_idx in range(ncell):
            bc = jnp.where(mask & (cell == c_idx), 1.0, 0.0)
            slabs.append(jnp.dot(hT, bc, preferred_element_type=jnp.float32))
        sstack = jnp.concatenate(slabs, axis=0)
        eT = jax.nn.relu(
            jnp.dot(wsoc, sstack, preferred_element_type=jnp.float32) + b_eff)

        gates = (jnp.dot(wih_r, rT, preferred_element_type=jnp.float32)
                 + jnp.dot(wih_e, eT, preferred_element_type=jnp.float32)
                 + jnp.dot(whh, hT, preferred_element_type=jnp.float32)
                 + b_lstm)
        i_g = jax.nn.sigmoid(gates[0:hid])
        f_g = jax.nn.sigmoid(gates[hid:2 * hid])
        g_g = jnp.tanh(gates[2 * hid:3 * hid])
        o_g = jax.nn.sigmoid(gates[3 * hid:4 * hid])
        c2 = f_g * cT + i_g * g_g
        h2 = o_g * jnp.tanh(c2)

        outT = (jnp.dot(wout, h2, preferred_element_type=jnp.float32)
                + bout_c) * pmf
        out_ref[pl.ds(f, 1), :, :] = outT[None]
        h_scr[:] = h2
        c_scr[:] = c2
        return carry

    lo = jnp.maximum(tob_ref[0] + 1, 0)
    hi = jnp.minimum(tpr_ref[0] + 1, T)
    jax.lax.fori_loop(lo, hi, step, 0)


def kernel(X, part_masks, all_h_t, all_c_t, W_in, b_in, W_soc, b_soc,
           W_ih, W_hh, b_ih, b_hh, W_out, b_out, T_obs, T_pred):
    T, N = X.shape[0], X.shape[1]
    hid = W_hh.shape[1]
    out_dim = W_out.shape[0]

    xt = jnp.transpose(X, (0, 2, 1))
    xc = X[:, :, 2:4]
    h0 = jnp.transpose(all_h_t)
    c0 = jnp.transpose(all_c_t)
    tob = jnp.asarray(T_obs, jnp.int32).reshape(1)
    tpr = jnp.asarray(T_pred, jnp.int32).reshape(1)

    smem = pl.BlockSpec(memory_space=pltpu.SMEM)

    out = pl.pallas_call(
        _social_lstm_body,
        out_shape=jax.ShapeDtypeStruct((T, out_dim, N), X.dtype),
        in_specs=[smem, smem] + [pl.BlockSpec()] * 15,
        out_specs=pl.BlockSpec(),
        scratch_shapes=[pltpu.VMEM((hid, N), jnp.float32),
                        pltpu.VMEM((hid, N), jnp.float32)],
    )(tob, tpr, xt, xc, part_masks, h0, c0,
      W_in, b_in.reshape(-1, 1), W_soc, b_soc.reshape(-1, 1),
      W_ih, W_hh, b_ih.reshape(-1, 1), b_hh.reshape(-1, 1),
      W_out, b_out.reshape(-1, 1))
    return jnp.transpose(out, (0, 2, 1))


# speedup vs baseline: 43.0371x; 1.1305x over previous
import jax
import jax.numpy as jnp
from jax.experimental import pallas as pl
from jax.experimental.pallas import tpu as pltpu

_N_SIZE = 4
_CELL = 1.0


def _col(b_ref):
    return jnp.transpose(b_ref[:], (1, 0))


def _social_lstm_body(tob_ref, tpr_ref, x_ref, pm_ref, h0_ref, c0_ref,
                      win_ref, bin_ref, wsoc_ref, bsoc_ref, wih_ref, whh_ref,
                      bih_ref, bhh_ref, wout_ref, bout_ref,
                      out_ref, h_scr, c_scr):
    T = out_ref.shape[0]
    N = out_ref.shape[1]
    med = win_ref.shape[0]
    hid = whh_ref.shape[1]
    ncell = _N_SIZE * _N_SIZE
    half = _N_SIZE / 2.0

    out_ref[:] = jnp.zeros(out_ref.shape, out_ref.dtype)
    h_scr[:] = jnp.transpose(h0_ref[:], (1, 0))
    c_scr[:] = jnp.transpose(c0_ref[:], (1, 0))

    wsoc = wsoc_ref[:]
    b_eff = _col(bsoc_ref) + jnp.sum(wsoc, axis=1, keepdims=True)
    b_lstm = _col(bih_ref) + _col(bhh_ref)
    bin_c = _col(bin_ref)
    bout_c = _col(bout_ref)
    wih_r = wih_ref[:, 0:med]
    wih_e = wih_ref[:, med:]
    whh = whh_ref[:]
    wout = wout_ref[:]
    win = win_ref[:]

    iota_l = jax.lax.broadcasted_iota(jnp.int32, (N, N), 1)
    iota_s = jax.lax.broadcasted_iota(jnp.int32, (N, N), 0)
    noteye = iota_l != iota_s

    def step(f, carry):
        xf = x_ref[pl.ds(f, 1)][0]
        pmf = pm_ref[pl.ds(f, 1)][0]
        hT = h_scr[:]
        cT = c_scr[:]

        xcol = xf[:, 2:4]
        xrow = jnp.transpose(xcol, (1, 0))
        cx_c = xcol[:, 0:1]
        cy_c = xcol[:, 1:2]
        cx_r = xrow[0:1, :]
        cy_r = xrow[1:2, :]

        gx = (cx_c - cx_r) / _CELL
        gy = (cy_c - cy_r) / _CELL
        inb = (jnp.abs(gx) <= half) & (jnp.abs(gy) <= half)
        mask = inb & noteye
        ix = jnp.clip(jnp.floor(gx + half).astype(jnp.int32), 0, _N_SIZE - 1)
        iy = jnp.clip(jnp.floor(gy + half).astype(jnp.int32), 0, _N_SIZE - 1)
        cell = ix * _N_SIZE + iy

        rT = jax.nn.relu(
            jnp.dot(win, xrow, preferred_element_type=jnp.float32)
            + bin_c)

        slabs = []
        for c_idx in range(ncell):
            bc = jnp.where(mask & (cell == c_idx), 1.0, 0.0)
            slabs.append(jnp.dot(hT, bc, preferred_element_type=jnp.float32))
        sstack = jnp.concatenate(slabs, axis=0)
        eT = jax.nn.relu(
            jnp.dot(wsoc, sstack, preferred_element_type=jnp.float32) + b_eff)

        gates = (jnp.dot(wih_r, rT, preferred_element_type=jnp.float32)
                 + jnp.dot(wih_e, eT, preferred_element_type=jnp.float32)
                 + jnp.dot(whh, hT, preferred_element_type=jnp.float32)
                 + b_lstm)
        i_g = jax.nn.sigmoid(gates[0:hid])
        f_g = jax.nn.sigmoid(gates[hid:2 * hid])
        g_g = jnp.tanh(gates[2 * hid:3 * hid])
        o_g = jax.nn.sigmoid(gates[3 * hid:4 * hid])
        c2 = f_g * cT + i_g * g_g
        h2 = o_g * jnp.tanh(c2)

        outT = (jnp.dot(wout, h2, preferred_element_type=jnp.float32)
                + bout_c) * pmf
        out_ref[pl.ds(f, 1), :, :] = jnp.transpose(outT, (1, 0))[None]
        h_scr[:] = h2
        c_scr[:] = c2
        return carry

    lo = jnp.maximum(tob_ref[0] + 1, 0)
    hi = jnp.minimum(tpr_ref[0] + 1, T)
    jax.lax.fori_loop(lo, hi, step, 0)


def kernel(X, part_masks, all_h_t, all_c_t, W_in, b_in, W_soc, b_soc,
           W_ih, W_hh, b_ih, b_hh, W_out, b_out, T_obs, T_pred):
    T, N = X.shape[0], X.shape[1]
    hid = W_hh.shape[1]
    out_dim = W_out.shape[0]

    tob = jnp.asarray(T_obs, jnp.int32).reshape(1)
    tpr = jnp.asarray(T_pred, jnp.int32).reshape(1)

    smem = pl.BlockSpec(memory_space=pltpu.SMEM)

    return pl.pallas_call(
        _social_lstm_body,
        out_shape=jax.ShapeDtypeStruct((T, N, out_dim), X.dtype),
        in_specs=[smem, smem] + [pl.BlockSpec()] * 14,
        out_specs=pl.BlockSpec(),
        scratch_shapes=[pltpu.VMEM((hid, N), jnp.float32),
                        pltpu.VMEM((hid, N), jnp.float32)],
    )(tob, tpr, X, part_masks, all_h_t, all_c_t,
      W_in, b_in.reshape(1, -1), W_soc, b_soc.reshape(1, -1),
      W_ih, W_hh, b_ih.reshape(1, -1), b_hh.reshape(1, -1),
      W_out, b_out.reshape(1, -1))


# speedup vs baseline: 43.2399x; 1.0047x over previous
import jax
import jax.numpy as jnp
from jax.experimental import pallas as pl
from jax.experimental.pallas import tpu as pltpu

_N_SIZE = 4
_CELL = 1.0


def _col(b_ref):
    return jnp.transpose(b_ref[:], (1, 0))


def _social_lstm_body(tob_ref, tpr_ref, x_ref, pm_ref, h0_ref, c0_ref,
                      win_ref, bin_ref, wsoc_ref, bsoc_ref, wih_ref, whh_ref,
                      bih_ref, bhh_ref, wout_ref, bout_ref,
                      out_ref, h_scr, c_scr):
    T = out_ref.shape[0]
    N = out_ref.shape[1]
    med = win_ref.shape[0]
    hid = whh_ref.shape[1]
    ncell = _N_SIZE * _N_SIZE
    half = _N_SIZE / 2.0

    out_ref[:] = jnp.zeros(out_ref.shape, out_ref.dtype)
    h_scr[:] = jnp.transpose(h0_ref[:], (1, 0))
    c_scr[:] = jnp.transpose(c0_ref[:], (1, 0))

    wsoc = wsoc_ref[:]
    b_eff = _col(bsoc_ref) + jnp.sum(wsoc, axis=1, keepdims=True)
    b_lstm = _col(bih_ref) + _col(bhh_ref)
    bin_c = _col(bin_ref)
    bout_c = _col(bout_ref)
    wih_r = wih_ref[:, 0:med]
    wih_e = wih_ref[:, med:]
    whh = whh_ref[:]
    wout = wout_ref[:]
    win = win_ref[:]

    iota_l = jax.lax.broadcasted_iota(jnp.int32, (N, N), 1)
    iota_s = jax.lax.broadcasted_iota(jnp.int32, (N, N), 0)
    noteye = iota_l != iota_s

    def step(f, carry):
        xf = x_ref[pl.ds(f, 1)][0]
        pmf = pm_ref[pl.ds(f, 1)][0]
        hT = h_scr[:]
        cT = c_scr[:]

        xcol = xf[:, 2:4]
        xrow = jnp.transpose(xcol, (1, 0))
        cx_c = xcol[:, 0:1]
        cy_c = xcol[:, 1:2]
        cx_r = xrow[0:1, :]
        cy_r = xrow[1:2, :]

        gx = (cx_c - cx_r) / _CELL
        gy = (cy_c - cy_r) / _CELL
        inb = (jnp.abs(gx) <= half) & (jnp.abs(gy) <= half)
        mask = inb & noteye
        ix = jnp.clip(jnp.floor(gx + half).astype(jnp.int32), 0, _N_SIZE - 1)
        iy = jnp.clip(jnp.floor(gy + half).astype(jnp.int32), 0, _N_SIZE - 1)
        cell = ix * _N_SIZE + iy
        cm = jnp.where(mask, cell, ncell)

        rT = jax.nn.relu(
            jnp.dot(win, xrow, preferred_element_type=jnp.float32)
            + bin_c)

        e_pre = b_eff
        for c_idx in range(ncell):
            bc = (cm == c_idx).astype(jnp.float32)
            sc = jnp.dot(hT, bc, preferred_element_type=jnp.float32)
            e_pre = e_pre + jnp.dot(
                wsoc[:, c_idx * hid:(c_idx + 1) * hid], sc,
                preferred_element_type=jnp.float32)
        eT = jax.nn.relu(e_pre)

        gates = (jnp.dot(wih_r, rT, preferred_element_type=jnp.float32)
                 + jnp.dot(wih_e, eT, preferred_element_type=jnp.float32)
                 + jnp.dot(whh, hT, preferred_element_type=jnp.float32)
                 + b_lstm)
        i_g = jax.nn.sigmoid(gates[0:hid])
        f_g = jax.nn.sigmoid(gates[hid:2 * hid])
        g_g = jnp.tanh(gates[2 * hid:3 * hid])
        o_g = jax.nn.sigmoid(gates[3 * hid:4 * hid])
        c2 = f_g * cT + i_g * g_g
        h2 = o_g * jnp.tanh(c2)

        outT = (jnp.dot(wout, h2, preferred_element_type=jnp.float32)
                + bout_c) * pmf
        out_ref[pl.ds(f, 1), :, :] = jnp.transpose(outT, (1, 0))[None]
        h_scr[:] = h2
        c_scr[:] = c2
        return carry

    lo = jnp.maximum(tob_ref[0] + 1, 0)
    hi = jnp.minimum(tpr_ref[0] + 1, T)
    jax.lax.fori_loop(lo, hi, step, 0)


def kernel(X, part_masks, all_h_t, all_c_t, W_in, b_in, W_soc, b_soc,
           W_ih, W_hh, b_ih, b_hh, W_out, b_out, T_obs, T_pred):
    T, N = X.shape[0], X.shape[1]
    hid = W_hh.shape[1]
    out_dim = W_out.shape[0]

    tob = jnp.asarray(T_obs, jnp.int32).reshape(1)
    tpr = jnp.asarray(T_pred, jnp.int32).reshape(1)

    smem = pl.BlockSpec(memory_space=pltpu.SMEM)

    return pl.pallas_call(
        _social_lstm_body,
        out_shape=jax.ShapeDtypeStruct((T, N, out_dim), X.dtype),
        in_specs=[smem, smem] + [pl.BlockSpec()] * 14,
        out_specs=pl.BlockSpec(),
        scratch_shapes=[pltpu.VMEM((hid, N), jnp.float32),
                        pltpu.VMEM((hid, N), jnp.float32)],
    )(tob, tpr, X, part_masks, all_h_t, all_c_t,
      W_in, b_in.reshape(1, -1), W_soc, b_soc.reshape(1, -1),
      W_ih, W_hh, b_ih.reshape(1, -1), b_hh.reshape(1, -1),
      W_out, b_out.reshape(1, -1))
